# Initial kernel scaffold; baseline (speedup 1.0000x reference)
import jax
import jax.numpy as jnp
from jax.experimental import pallas as pl


def kernel(x, edge_index, batch, W_root0, W_rel0, b0, W_root1, W_rel1, b1, W_root2, W_rel2, b2, Wh1, bh1, Wh2, bh2):
    raise NotImplementedError("write your pallas kernel here")


# speedup vs baseline: 6.1652x; 6.1652x over previous
import functools

import jax
import jax.numpy as jnp
from jax import lax
from jax.experimental import pallas as pl
from jax.experimental.pallas import tpu as pltpu
from jax.experimental.pallas import tpu_sc as plsc

N, E, D, H, G = 10000, 320000, 128, 64, 64

NC, NS = 2, 16
NW = NC * NS
CH = 128
EP_W = -(-E // (NW * CH)) * CH
NCHUNK = EP_W // CH
E_PAD = NW * EP_W
ZR = -(-(N + 1) // (NS * 8)) * 8
N_ACC = ZR * NS


def _sc_body(p_hbm, src_hbm, dst_hbm, zeros_hbm, out_hbm,
             src_v, dst_v, rows_v, acc_sh, sem):
    c = lax.axis_index("c")
    s = lax.axis_index("s")
    wid = c * NS + s

    pltpu.sync_copy(src_hbm.at[wid], src_v)
    pltpu.sync_copy(dst_hbm.at[wid], dst_v)

    pltpu.sync_copy(zeros_hbm, acc_sh.at[pl.ds(s * ZR, ZR)])
    plsc.subcore_barrier()

    def chunk(j, carry):
        pltpu.async_copy(p_hbm.at[src_v.at[j]], rows_v, sem).wait()
        pltpu.sync_copy(rows_v, acc_sh.at[dst_v.at[j]], add=True)
        return carry

    lax.fori_loop(0, NCHUNK, chunk, 0)
    plsc.subcore_barrier()
    pltpu.sync_copy(acc_sh.at[pl.ds(s * ZR, ZR)],
                    out_hbm.at[pl.ds(c * N_ACC + s * ZR, ZR)])


@functools.cache
def _sc_segment_sum():
    return pl.kernel(
        _sc_body,
        out_type=jax.ShapeDtypeStruct((NC * N_ACC, H), jnp.float32),
        mesh=plsc.VectorSubcoreMesh(core_axis_name="c", subcore_axis_name="s",
                                    num_cores=NC, num_subcores=NS),
        scratch_types=[
            pltpu.VMEM((NCHUNK, CH), jnp.int32),
            pltpu.VMEM((NCHUNK, CH), jnp.int32),
            pltpu.VMEM((CH, H), jnp.float32),
            pltpu.VMEM_SHARED((N_ACC, H), jnp.float32),
            pltpu.SemaphoreType.DMA,
        ],
        compiler_params=pltpu.CompilerParams(use_tc_tiling_on_sc=False),
    )


def _pre_body(x_ref, wr_ref, wn_ref, b_ref, p_ref, r_ref):
    x = x_ref[...]
    p_ref[...] = jnp.dot(x, wn_ref[...], preferred_element_type=jnp.float32)
    r_ref[...] = (jnp.dot(x, wr_ref[...], preferred_element_type=jnp.float32)
                  + b_ref[...])


def _mid_body(r_ref, acc_ref, wr_ref, wn_ref, b_ref, p_ref, rout_ref):
    h = jnp.maximum(
        r_ref[...] + acc_ref[0:N, :] + acc_ref[N_ACC:N_ACC + N, :], 0.0)
    p_ref[...] = jnp.dot(h, wn_ref[...], preferred_element_type=jnp.float32)
    rout_ref[...] = (jnp.dot(h, wr_ref[...], preferred_element_type=jnp.float32)
                     + b_ref[...])


def _final_body(r_ref, acc_ref, batch_ref, wh1_ref, bh1_ref, wh2_ref, bh2_ref,
                out_ref):
    h = jnp.maximum(
        r_ref[...] + acc_ref[0:N, :] + acc_ref[N_ACC:N_ACC + N, :], 0.0)
    gid = lax.broadcasted_iota(jnp.int32, (G, N), 0)
    m = (gid == batch_ref[...]).astype(jnp.float32)
    sums = jnp.dot(m, h, preferred_element_type=jnp.float32)
    cnt = jnp.sum(m, axis=1, keepdims=True)
    pooled = sums / jnp.maximum(cnt, 1.0)
    z = jnp.maximum(
        jnp.dot(pooled, wh1_ref[...], preferred_element_type=jnp.float32)
        + bh1_ref[...], 0.0)
    out_ref[...] = (jnp.dot(z, wh2_ref[...], preferred_element_type=jnp.float32)
                    + bh2_ref[...])


def _tc_call(body, out_shapes):
    return pl.pallas_call(body, out_shape=out_shapes)


def kernel(x, edge_index, batch, W_root0, W_rel0, b0, W_root1, W_rel1, b1,
           W_root2, W_rel2, b2, Wh1, bh1, Wh2, bh2):
    f32 = jnp.float32
    src = jnp.concatenate(
        [edge_index[0], jnp.zeros((E_PAD - E,), jnp.int32)]).reshape(
            NW, NCHUNK, CH)
    dst = jnp.concatenate(
        [edge_index[1], jnp.full((E_PAD - E,), N, jnp.int32)]).reshape(
            NW, NCHUNK, CH)
    zeros_tile = jnp.zeros((ZR, H), f32)
    batch_row = batch.astype(jnp.int32).reshape(1, N)

    nh = jax.ShapeDtypeStruct((N, H), f32)
    pre = _tc_call(_pre_body, (nh, nh))
    mid = _tc_call(_mid_body, (nh, nh))
    final = _tc_call(_final_body, jax.ShapeDtypeStruct((G, 128), f32))

    sc_seg = _sc_segment_sum()
    p0, r0 = pre(x, W_root0, W_rel0, b0.reshape(1, H))
    acc0 = sc_seg(p0, src, dst, zeros_tile)
    p1, r1 = mid(r0, acc0, W_root1, W_rel1, b1.reshape(1, H))
    acc1 = sc_seg(p1, src, dst, zeros_tile)
    p2, r2 = mid(r1, acc1, W_root2, W_rel2, b2.reshape(1, H))
    acc2 = sc_seg(p2, src, dst, zeros_tile)

    wh2_pad = jnp.zeros((H, 128), f32).at[:, :2].set(Wh2)
    bh2_pad = jnp.zeros((1, 128), f32).at[0, :2].set(bh2)
    out_pad = final(r2, acc2, batch_row, Wh1, bh1.reshape(1, H),
                    wh2_pad, bh2_pad)
    return out_pad[:, :2]
